# baseline (speedup 1.0000x reference)
import jax
import jax.numpy as jnp
from jax.experimental import pallas as pl
from jax.experimental.pallas import tpu as pltpu

_R = 48


def _sum_body(x_ref, out_ref):
    s = jnp.sum(x_ref[...], axis=1, keepdims=True)
    out_ref[...] = jnp.broadcast_to(s, out_ref.shape)


def _mlp_topk_body(s_ref, w1_ref, b1_ref, w2_ref, b2_ref, idx_ref, *, n_spatial):
    b, c = s_ref.shape
    r = _R
    y0 = s_ref[...] * (1.0 / n_spatial)
    h = jax.lax.dot_general(y0, w1_ref[...], (((1,), (1,)), ((), ())),
                            preferred_element_type=jnp.float32) + b1_ref[...]
    h = jnp.where(h >= 0, h, 0.01 * h)
    z = jax.lax.dot_general(h, w2_ref[...], (((1,), (1,)), ((), ())),
                            preferred_element_type=jnp.float32) + b2_ref[...]
    s = jax.nn.sigmoid(z)
    si = s[:, :, None]
    sj = s[:, None, :]
    ii = jax.lax.broadcasted_iota(jnp.int32, (b, c, c), 1)
    jj = jax.lax.broadcasted_iota(jnp.int32, (b, c, c), 2)
    beats = (sj > si) | ((sj == si) & (jj < ii))
    rank = jnp.sum(beats.astype(jnp.int32), axis=2)
    pp = jax.lax.broadcasted_iota(jnp.int32, (b, c, r), 2)
    im = jax.lax.broadcasted_iota(jnp.int32, (b, c, r), 1)
    onehot = (rank[:, :, None] == pp).astype(jnp.int32)
    idx_ref[...] = jnp.sum(onehot * im, axis=1)


def _gather_body(idx_ref, x_ref, o_ref):
    del idx_ref
    o_ref[...] = x_ref[...]


def kernel(x, w1, b1, w2, b2):
    b, c, d, h, w = x.shape
    n = d * h * w
    rows = b * c
    row_blk = 16
    xr = x.reshape(rows, n)

    sums = pl.pallas_call(
        _sum_body,
        grid=(rows // row_blk,),
        in_specs=[pl.BlockSpec((row_blk, n), lambda i: (i, 0))],
        out_specs=pl.BlockSpec((row_blk, 128), lambda i: (i, 0)),
        out_shape=jax.ShapeDtypeStruct((rows, 128), jnp.float32),
    )(xr)

    import functools
    idx = pl.pallas_call(
        functools.partial(_mlp_topk_body, n_spatial=n),
        out_shape=jax.ShapeDtypeStruct((b, _R), jnp.int32),
    )(sums[:, 0].reshape(b, c), w1, b1.reshape(1, c), w2, b2.reshape(1, c))

    xg = x.reshape(b, c, n // 128, 128)
    out = pl.pallas_call(
        _gather_body,
        grid_spec=pltpu.PrefetchScalarGridSpec(
            num_scalar_prefetch=1,
            grid=(b, _R),
            in_specs=[pl.BlockSpec(
                (1, 1, n // 128, 128),
                lambda bi, ri, idx_ref: (bi, idx_ref[bi, ri], 0, 0))],
            out_specs=pl.BlockSpec(
                (1, 1, n // 128, 128),
                lambda bi, ri, idx_ref: (bi, ri, 0, 0)),
        ),
        out_shape=jax.ShapeDtypeStruct((b, _R, n // 128, 128), jnp.float32),
    )(idx, xg)
    return out.reshape(b, _R, d, h, w)


# speedup vs baseline: 1.5175x; 1.5175x over previous
import functools

import jax
import jax.numpy as jnp
from jax.experimental import pallas as pl
from jax.experimental.pallas import tpu as pltpu

_R = 48


def _sum_body(x_ref, out_ref):
    out_ref[0] = jnp.sum(x_ref[0], axis=(1, 2))


def _mlp_topk_body(s_ref, w1_ref, b1_ref, w2_ref, b2_ref, idx_ref, *, n_spatial):
    b, c, _ = s_ref.shape
    r = _R
    y0 = jnp.sum(s_ref[...], axis=2) * (1.0 / n_spatial)
    h = jax.lax.dot_general(y0, w1_ref[...], (((1,), (1,)), ((), ())),
                            preferred_element_type=jnp.float32) + b1_ref[...]
    h = jnp.where(h >= 0, h, 0.01 * h)
    z = jax.lax.dot_general(h, w2_ref[...], (((1,), (1,)), ((), ())),
                            preferred_element_type=jnp.float32) + b2_ref[...]
    s = jax.nn.sigmoid(z)
    si = s[:, :, None]
    sj = s[:, None, :]
    ii = jax.lax.broadcasted_iota(jnp.int32, (b, c, c), 1)
    jj = jax.lax.broadcasted_iota(jnp.int32, (b, c, c), 2)
    beats = (sj > si) | ((sj == si) & (jj < ii))
    rank = jnp.sum(beats.astype(jnp.int32), axis=2)
    pp = jax.lax.broadcasted_iota(jnp.int32, (b, c, r), 2)
    im = jax.lax.broadcasted_iota(jnp.int32, (b, c, r), 1)
    onehot = (rank[:, :, None] == pp).astype(jnp.int32)
    idx_ref[...] = jnp.sum(onehot * im, axis=1)


def _gather_body(idx_ref, x_ref, o_ref):
    del idx_ref
    o_ref[...] = x_ref[...]


def kernel(x, w1, b1, w2, b2):
    b, c, d, h, w = x.shape
    n = d * h * w
    c_blk = 8

    sums = pl.pallas_call(
        _sum_body,
        grid=(b, c // c_blk),
        in_specs=[pl.BlockSpec((1, c_blk, d, h, w), lambda bi, ci: (bi, ci, 0, 0, 0))],
        out_specs=pl.BlockSpec((1, c_blk, w), lambda bi, ci: (bi, ci, 0)),
        out_shape=jax.ShapeDtypeStruct((b, c, w), jnp.float32),
    )(x)

    idx = pl.pallas_call(
        functools.partial(_mlp_topk_body, n_spatial=n),
        out_shape=jax.ShapeDtypeStruct((b, _R), jnp.int32),
    )(sums, w1, b1.reshape(1, c), w2, b2.reshape(1, c))

    out = pl.pallas_call(
        _gather_body,
        grid_spec=pltpu.PrefetchScalarGridSpec(
            num_scalar_prefetch=1,
            grid=(b, _R),
            in_specs=[pl.BlockSpec(
                (1, 1, d, h, w),
                lambda bi, ri, idx_ref: (bi, idx_ref[bi, ri], 0, 0, 0))],
            out_specs=pl.BlockSpec(
                (1, 1, d, h, w),
                lambda bi, ri, idx_ref: (bi, ri, 0, 0, 0)),
        ),
        out_shape=jax.ShapeDtypeStruct((b, _R, d, h, w), jnp.float32),
    )(idx, x)
    return out


# speedup vs baseline: 2.6217x; 1.7277x over previous
import functools

import jax
import jax.numpy as jnp
from jax.experimental import pallas as pl
from jax.experimental.pallas import tpu as pltpu

_R = 48


def _sum_body(x_ref, out_ref):
    si = pl.program_id(1)
    s = jnp.sum(x_ref[0], axis=0, keepdims=True)
    acc = jnp.broadcast_to(s, out_ref.shape[1:])

    @pl.when(si == 0)
    def _init():
        out_ref[0] = acc

    @pl.when(si != 0)
    def _acc():
        out_ref[0] += acc


def _mlp_topk_body(s_ref, w1_ref, b1_ref, w2_ref, b2_ref, p_ref, *, n_spatial):
    b, _, c = s_ref.shape
    r = _R
    y0 = s_ref[:, 0, :] * (1.0 / n_spatial)
    h = jax.lax.dot_general(y0, w1_ref[...], (((1,), (1,)), ((), ())),
                            preferred_element_type=jnp.float32) + b1_ref[...]
    h = jnp.where(h >= 0, h, 0.01 * h)
    z = jax.lax.dot_general(h, w2_ref[...], (((1,), (1,)), ((), ())),
                            preferred_element_type=jnp.float32) + b2_ref[...]
    s = jax.nn.sigmoid(z)
    si = s[:, :, None]
    sj = s[:, None, :]
    ii = jax.lax.broadcasted_iota(jnp.int32, (b, c, c), 1)
    jj = jax.lax.broadcasted_iota(jnp.int32, (b, c, c), 2)
    beats = (sj > si) | ((sj == si) & (jj < ii))
    rank = jnp.sum(beats.astype(jnp.int32), axis=2)
    pp = jax.lax.broadcasted_iota(jnp.int32, (b, c, r), 2)
    p_ref[...] = (rank[:, :, None] == pp).astype(jnp.float32)


def _gather_mm_body(x_ref, p_ref, o_ref):
    o_ref[0] = jax.lax.dot_general(
        x_ref[0], p_ref[0], (((1,), (0,)), ((), ())),
        preferred_element_type=jnp.float32)


def kernel(x, w1, b1, w2, b2):
    b, c, d, h, w = x.shape
    n = d * h * w
    xm = jnp.transpose(x, (0, 2, 3, 4, 1)).reshape(b, n, c)

    m_blk1 = 16384
    sums = pl.pallas_call(
        _sum_body,
        grid=(b, n // m_blk1),
        in_specs=[pl.BlockSpec((1, m_blk1, c), lambda bi, si: (bi, si, 0))],
        out_specs=pl.BlockSpec((1, 8, c), lambda bi, si: (bi, 0, 0)),
        out_shape=jax.ShapeDtypeStruct((b, 8, c), jnp.float32),
    )(xm)

    pmat = pl.pallas_call(
        functools.partial(_mlp_topk_body, n_spatial=n),
        out_shape=jax.ShapeDtypeStruct((b, c, _R), jnp.float32),
    )(sums, w1, b1.reshape(1, c), w2, b2.reshape(1, c))

    m_blk2 = 8192
    out_t = pl.pallas_call(
        _gather_mm_body,
        grid=(b, n // m_blk2),
        in_specs=[
            pl.BlockSpec((1, m_blk2, c), lambda bi, si: (bi, si, 0)),
            pl.BlockSpec((1, c, _R), lambda bi, si: (bi, 0, 0)),
        ],
        out_specs=pl.BlockSpec((1, m_blk2, _R), lambda bi, si: (bi, si, 0)),
        out_shape=jax.ShapeDtypeStruct((b, n, _R), jnp.float32),
    )(xm, pmat)

    return jnp.transpose(out_t.reshape(b, d, h, w, _R), (0, 4, 1, 2, 3))
